# baseline (speedup 1.0000x reference)
import functools
import math

import jax
import jax.numpy as jnp
from jax import lax
from jax.experimental import pallas as pl
from jax.experimental.pallas import tpu as pltpu
from jax.experimental.pallas import tpu_sc as plsc

D_EMBED = 64
SCALE = math.sqrt(D_EMBED)
LANES = 16


def _embed_flat(idx_flat, table):
    n = idx_flat.shape[0]
    info = plsc.get_sparse_core_info()
    num_workers = info.num_cores * info.num_subcores
    per_w = n // num_workers
    chunk = 512
    n_chunks = per_w // chunk
    vecs_per_row = D_EMBED // LANES

    mesh = plsc.VectorSubcoreMesh(core_axis_name="c", subcore_axis_name="s")

    @functools.partial(
        pl.kernel,
        mesh=mesh,
        out_type=jax.ShapeDtypeStruct((n, D_EMBED), jnp.float32),
        scratch_types=[
            pltpu.VMEM((chunk,), jnp.int32),
            pltpu.VMEM((chunk, D_EMBED), jnp.float32),
            pltpu.SemaphoreType.DMA,
        ],
        compiler_params=pltpu.CompilerParams(use_tc_tiling_on_sc=False),
    )
    def emb(idx_hbm, table_hbm, out_hbm, idx_v, rows_v, sem):
        wid = lax.axis_index("s") * info.num_cores + lax.axis_index("c")
        base = wid * per_w

        def chunk_body(g, carry):
            off = base + g * chunk
            pltpu.sync_copy(idx_hbm.at[pl.ds(off, chunk)], idx_v)
            pltpu.async_copy(table_hbm.at[idx_v], rows_v, sem).wait()

            def scale_body(i, c):
                for j in range(vecs_per_row):
                    rows_v[i, pl.ds(j * LANES, LANES)] = (
                        rows_v[i, pl.ds(j * LANES, LANES)] * SCALE
                    )
                return c

            lax.fori_loop(0, chunk, scale_body, 0)
            pltpu.sync_copy(rows_v, out_hbm.at[pl.ds(off, chunk)])
            return carry

        lax.fori_loop(0, n_chunks, chunk_body, 0)

    return emb(idx_flat, table)


def kernel(x, table):
    idx_flat = x.reshape(-1).astype(jnp.int32)
    out = _embed_flat(idx_flat, table)
    return out.reshape(x.shape[0], x.shape[1], D_EMBED)


# speedup vs baseline: 1.4286x; 1.4286x over previous
import functools
import math

import jax
import jax.numpy as jnp
from jax import lax
from jax.experimental import pallas as pl
from jax.experimental.pallas import tpu as pltpu
from jax.experimental.pallas import tpu_sc as plsc

D_EMBED = 64
SCALE = math.sqrt(D_EMBED)
LANES = 16


def _gather_scale(idx_flat, table):
    n = idx_flat.shape[0]
    info = plsc.get_sparse_core_info()
    num_workers = info.num_cores * info.num_subcores
    per_w = n // num_workers
    chunk = 512
    n_chunks = per_w // chunk
    vecs_per_row = D_EMBED // LANES

    mesh = plsc.VectorSubcoreMesh(core_axis_name="c", subcore_axis_name="s")

    @functools.partial(
        pl.kernel,
        mesh=mesh,
        out_type=jax.ShapeDtypeStruct((n, D_EMBED), jnp.float32),
        scratch_types=[
            pltpu.VMEM((chunk,), jnp.int32),
            pltpu.VMEM((chunk, D_EMBED), jnp.float32),
            pltpu.SemaphoreType.DMA,
        ],
    )
    def emb(idx_hbm, table_hbm, out_hbm, idx_v, rows_v, sem):
        wid = lax.axis_index("s") * info.num_cores + lax.axis_index("c")
        base = wid * per_w

        def chunk_body(g, carry):
            off = base + g * chunk
            pltpu.sync_copy(idx_hbm.at[pl.ds(off, chunk)], idx_v)

            def fire_body(k, c):
                vec = idx_v[pl.ds(k * LANES, LANES)]
                for j in range(LANES):
                    pltpu.async_copy(
                        table_hbm.at[vec[j]],
                        rows_v.at[k * LANES + j],
                        sem,
                    )
                return c

            lax.fori_loop(0, chunk // LANES, fire_body, 0)
            pltpu.make_async_copy(
                table_hbm.at[pl.ds(0, chunk)], rows_v, sem
            ).wait()

            def scale_body(i, c):
                for j in range(vecs_per_row):
                    rows_v[i, pl.ds(j * LANES, LANES)] = (
                        rows_v[i, pl.ds(j * LANES, LANES)] * SCALE
                    )
                return c

            lax.fori_loop(0, chunk, scale_body, 0)
            pltpu.sync_copy(rows_v, out_hbm.at[pl.ds(off, chunk)])
            return carry

        lax.fori_loop(0, n_chunks, chunk_body, 0)

    return emb(idx_flat, table)


def kernel(x, table):
    batch, seq = x.shape
    idx_flat = x.reshape(-1).astype(jnp.int32)
    out = _gather_scale(idx_flat, table)
    return out.reshape(batch, seq, D_EMBED)
